# Initial kernel scaffold; baseline (speedup 1.0000x reference)
import jax
import jax.numpy as jnp
from jax.experimental import pallas as pl


def kernel(failure_probability, failure_label, voltages, angles, line_flows, frequency, target_voltages, conductance, susceptance, power_injection, thermal_limits, reactive_injection, power_imbalance, edge_index):
    raise NotImplementedError("write your pallas kernel here")


# speedup vs baseline: 1.0471x; 1.0471x over previous
import jax
import jax.numpy as jnp
from jax import lax
from jax.experimental import pallas as pl
from jax.experimental.pallas import tpu as pltpu

_B, _N, _E = 2, 100000, 3200000
_NPAD = 102400
_WN = 12800
_WE = 128000

_INTERPRET = False


def _node_loss_kernel(fp, fl, v, tgt, pinj, qinj, pp, qp, freq, pimb, out):
    i = pl.program_id(0)
    col = lax.broadcasted_iota(jnp.int32, (1, _WN), 1) + i * _WN
    m = col < _N

    @pl.when(i == 0)
    def _init():
        for k in range(5):
            out[k] = 0.0
        ef = 60.0 + 6.0 * pimb[...]
        out[5] = jnp.sum((freq[...] - ef) ** 2)

    p = jnp.clip(fp[...], 1e-6, 1.0 - 1e-6)
    bce = -(fl[...] * jnp.log(p) + (1.0 - fl[...]) * jnp.log(1.0 - p))
    out[0] += jnp.sum(jnp.where(m, bce, 0.0))

    p_calc = jnp.sum(pp[...], axis=0)
    out[1] += jnp.sum(jnp.where(m, (p_calc - pinj[...]) ** 2, 0.0))
    q_calc = jnp.sum(qp[...], axis=0)
    out[2] += jnp.sum(jnp.where(m, (q_calc - qinj[...]) ** 2, 0.0))

    vv = v[...]
    low = jnp.maximum(0.95 - vv, 0.0)
    high = jnp.maximum(vv - 1.05, 0.0)
    out[3] += jnp.sum(jnp.where(m, low * low + high * high, 0.0))
    out[4] += jnp.sum(jnp.where(m, (vv - tgt[...]) ** 2, 0.0))


def _edge_loss_kernel(lf, tl, out):
    i = pl.program_id(0)

    @pl.when(i == 0)
    def _init():
        out[0] = 0.0

    viol = jnp.maximum(jnp.abs(lf[...]) - tl[...], 0.0)
    out[0] += jnp.sum(viol * viol)


def _prep_kernel(v, th, ab):
    vv = v[...]
    t = th[...]
    ab[0] = vv * jnp.cos(t)
    ab[1] = vv * jnp.sin(t)


def _run_prep(vpad, apad):
    return pl.pallas_call(
        _prep_kernel,
        out_shape=jax.ShapeDtypeStruct((2, _B, _NPAD), jnp.float32),
        interpret=_INTERPRET,
    )(vpad, apad)


def _run_node_loss(fp, fl, v, tgt, pinj, qinj, pp, qp, freq, pimb):
    nc = pp.shape[0]
    spec_n = pl.BlockSpec((_B, _WN), lambda i: (0, i))
    spec_p = pl.BlockSpec((nc, _B, _WN), lambda i: (0, 0, i))
    spec_s = pl.BlockSpec((_B, 1), lambda i: (0, 0))
    return pl.pallas_call(
        _node_loss_kernel,
        grid=(_NPAD // _WN,),
        in_specs=[spec_n] * 6 + [spec_p, spec_p, spec_s, spec_s],
        out_specs=pl.BlockSpec(memory_space=pltpu.SMEM),
        out_shape=jax.ShapeDtypeStruct((6,), jnp.float32),
        interpret=_INTERPRET,
    )(fp, fl, v, tgt, pinj, qinj, pp, qp, freq, pimb)


def _run_edge_loss(lf, tl):
    spec_e = pl.BlockSpec((_B, _WE), lambda i: (0, i))
    return pl.pallas_call(
        _edge_loss_kernel,
        grid=(_E // _WE,),
        in_specs=[spec_e, spec_e],
        out_specs=pl.BlockSpec(memory_space=pltpu.SMEM),
        out_shape=jax.ShapeDtypeStruct((1,), jnp.float32),
        interpret=_INTERPRET,
    )(lf, tl)


def _flow_partials(ab, edge_index, conductance, susceptance):
    a = ab[0, :, :_N]
    b = ab[1, :, :_N]
    src = edge_index[0]
    dst = edge_index[1]
    ai = jnp.take(a, src, axis=1)
    aj = jnp.take(a, dst, axis=1)
    bi = jnp.take(b, src, axis=1)
    bj = jnp.take(b, dst, axis=1)
    c = ai * aj + bi * bj
    s = bi * aj - ai * bj
    p_ij = conductance * c + susceptance * s
    q_ij = conductance * s - susceptance * c
    p_calc = jnp.zeros((_B, _N), jnp.float32)
    p_calc = p_calc.at[:, src].add(p_ij)
    p_calc = p_calc.at[:, dst].add(-p_ij)
    q_calc = jnp.zeros((_B, _N), jnp.float32)
    q_calc = q_calc.at[:, src].add(q_ij)
    q_calc = q_calc.at[:, dst].add(-q_ij)
    pad = ((0, 0), (0, _NPAD - _N))
    return (jnp.pad(p_calc, pad)[None], jnp.pad(q_calc, pad)[None])


def kernel(failure_probability, failure_label, voltages, angles, line_flows,
           frequency, target_voltages, conductance, susceptance,
           power_injection, thermal_limits, reactive_injection,
           power_imbalance, edge_index):
    v = voltages[..., 0]
    th = angles[..., 0]
    fp = failure_probability[..., 0]
    fl = failure_label[..., 0]
    tgt = target_voltages[..., 0]
    pinj = power_injection[..., 0]
    qinj = reactive_injection[..., 0]
    lf = line_flows[..., 0]

    pad = ((0, 0), (0, _NPAD - _N))
    vpad = jnp.pad(v, pad)
    apad = jnp.pad(th, pad)
    ab = _run_prep(vpad, apad)

    pp, qp = _flow_partials(ab, edge_index, conductance, susceptance)

    sums = _run_node_loss(fp, fl, v, tgt, pinj, qinj, pp, qp,
                          frequency, power_imbalance)
    cap = _run_edge_loss(lf, thermal_limits)

    bn = float(_B * _N)
    total = (sums[0] / bn
             + 0.1 * (sums[1] / bn)
             + 0.05 * (cap[0] / float(_B * _E))
             + 0.05 * (sums[3] / bn)
             + 0.08 * (sums[5] / float(_B))
             + 1.0 * (sums[4] / bn)
             + 0.1 * (sums[2] / bn))
    return total


# speedup vs baseline: 81.3859x; 77.7245x over previous
import functools

import jax
import jax.numpy as jnp
from jax import lax
from jax.experimental import pallas as pl
from jax.experimental.pallas import tpu as pltpu
from jax.experimental.pallas import tpu_sc as plsc

_B, _N, _E = 2, 100000, 3200000
_NPAD = 102400
_WN = 12800
_WE = 128000

_NC, _NS = 2, 16
_NW = _NC * _NS
_EPW = _E // _NW
_K = 2000
_SLICE = _NPAD // _NS

_INTERPRET = False


def _node_loss_kernel(fp, fl, v, tgt, pinj, qinj, pp, qp, freq, pimb, out):
    i = pl.program_id(0)
    col = lax.broadcasted_iota(jnp.int32, (1, _WN), 1) + i * _WN
    m = col < _N

    @pl.when(i == 0)
    def _init():
        for k in range(5):
            out[k] = 0.0
        ef = 60.0 + 6.0 * pimb[...]
        out[5] = jnp.sum((freq[...] - ef) ** 2)

    p = jnp.clip(fp[...], 1e-6, 1.0 - 1e-6)
    bce = -(fl[...] * jnp.log(p) + (1.0 - fl[...]) * jnp.log(1.0 - p))
    out[0] += jnp.sum(jnp.where(m, bce, 0.0))

    p_calc = jnp.sum(pp[...], axis=0)
    out[1] += jnp.sum(jnp.where(m, (p_calc - pinj[...]) ** 2, 0.0))
    q_calc = jnp.sum(qp[...], axis=0)
    out[2] += jnp.sum(jnp.where(m, (q_calc - qinj[...]) ** 2, 0.0))

    vv = v[...]
    low = jnp.maximum(0.95 - vv, 0.0)
    high = jnp.maximum(vv - 1.05, 0.0)
    out[3] += jnp.sum(jnp.where(m, low * low + high * high, 0.0))
    out[4] += jnp.sum(jnp.where(m, (vv - tgt[...]) ** 2, 0.0))


def _edge_loss_kernel(lf, tl, out):
    i = pl.program_id(0)

    @pl.when(i == 0)
    def _init():
        out[0] = 0.0

    viol = jnp.maximum(jnp.abs(lf[...]) - tl[...], 0.0)
    out[0] += jnp.sum(viol * viol)


def _prep_kernel(v, th, ab):
    vv = v[...]
    t = th[...]
    ab[0] = vv * jnp.cos(t)
    ab[1] = vv * jnp.sin(t)


def _run_prep(vpad, apad):
    return pl.pallas_call(
        _prep_kernel,
        out_shape=jax.ShapeDtypeStruct((2, _B, _NPAD), jnp.float32),
        interpret=_INTERPRET,
    )(vpad, apad)


def _run_node_loss(fp, fl, v, tgt, pinj, qinj, pp, qp, freq, pimb):
    nc = pp.shape[0]
    spec_n = pl.BlockSpec((_B, _WN), lambda i: (0, i))
    spec_p = pl.BlockSpec((nc, _B, _WN), lambda i: (0, 0, i))
    spec_s = pl.BlockSpec((_B, 1), lambda i: (0, 0))
    return pl.pallas_call(
        _node_loss_kernel,
        grid=(_NPAD // _WN,),
        in_specs=[spec_n] * 6 + [spec_p, spec_p, spec_s, spec_s],
        out_specs=pl.BlockSpec(memory_space=pltpu.SMEM),
        out_shape=jax.ShapeDtypeStruct((6,), jnp.float32),
        interpret=_INTERPRET,
    )(fp, fl, v, tgt, pinj, qinj, pp, qp, freq, pimb)


def _run_edge_loss(lf, tl):
    spec_e = pl.BlockSpec((_B, _WE), lambda i: (0, i))
    return pl.pallas_call(
        _edge_loss_kernel,
        grid=(_E // _WE,),
        in_specs=[spec_e, spec_e],
        out_specs=pl.BlockSpec(memory_space=pltpu.SMEM),
        out_shape=jax.ShapeDtypeStruct((1,), jnp.float32),
        interpret=_INTERPRET,
    )(lf, tl)


def _sc_flow_body(a0, a1, b0, b1, srcx, dstx, g0, g1, s0, s1,
                  pout, qout,
                  ta0, ta1, tb0, tb1, ap0, ap1, aq0, aq1,
                  srcv, dstv, gv, sv, aiv, ajv, biv, bjv,
                  pv, npv, qv, nqv, zv):
    c = lax.axis_index("c")
    s = lax.axis_index("s")
    wid = c * _NS + s
    sl = pl.ds(s * _SLICE, _SLICE)

    def zero16(r, _):
        zv[pl.ds(r * 16, 16)] = jnp.zeros((16,), jnp.float32)
        return _
    lax.fori_loop(0, _SLICE // 16, zero16, None)
    for acc in (ap0, ap1, aq0, aq1):
        pltpu.sync_copy(zv, acc.at[sl])
    for hbm, tab in ((a0, ta0), (a1, ta1), (b0, tb0), (b1, tb1)):
        pltpu.sync_copy(hbm.at[sl], tab.at[sl])
    plsc.subcore_barrier()

    def chunk(t, carry):
        e0 = wid * _EPW + t * _K
        pltpu.sync_copy(srcx.at[pl.ds(e0, _K)], srcv)
        pltpu.sync_copy(dstx.at[pl.ds(e0, _K)], dstv)
        for ta, tb, ap, aq, gh, sh in ((ta0, tb0, ap0, aq0, g0, s0),
                                       (ta1, tb1, ap1, aq1, g1, s1)):
            pltpu.sync_copy(gh.at[pl.ds(e0, _K)], gv)
            pltpu.sync_copy(sh.at[pl.ds(e0, _K)], sv)
            pltpu.sync_copy(ta.at[srcv], aiv)
            pltpu.sync_copy(ta.at[dstv], ajv)
            pltpu.sync_copy(tb.at[srcv], biv)
            pltpu.sync_copy(tb.at[dstv], bjv)

            def vec(r, carry2):
                ds = pl.ds(r * 16, 16)
                ai_, aj_ = aiv[ds], ajv[ds]
                bi_, bj_ = biv[ds], bjv[ds]
                cc = ai_ * aj_ + bi_ * bj_
                ss = bi_ * aj_ - ai_ * bj_
                g_, b_ = gv[ds], sv[ds]
                p = g_ * cc + b_ * ss
                q = g_ * ss - b_ * cc
                pv[ds] = p
                npv[ds] = -p
                qv[ds] = q
                nqv[ds] = -q
                return carry2
            lax.fori_loop(0, _K // 16, vec, None)
            pltpu.sync_copy(pv, ap.at[srcv], add=True)
            pltpu.sync_copy(npv, ap.at[dstv], add=True)
            pltpu.sync_copy(qv, aq.at[srcv], add=True)
            pltpu.sync_copy(nqv, aq.at[dstv], add=True)
        return carry
    lax.fori_loop(0, _EPW // _K, chunk, None)

    plsc.subcore_barrier()
    pltpu.sync_copy(ap0.at[sl], pout.at[c, 0, sl])
    pltpu.sync_copy(ap1.at[sl], pout.at[c, 1, sl])
    pltpu.sync_copy(aq0.at[sl], qout.at[c, 0, sl])
    pltpu.sync_copy(aq1.at[sl], qout.at[c, 1, sl])


def _run_sc_flow(a0, a1, b0, b1, srcx, dstx, g0, g1, s0, s1):
    mesh = plsc.VectorSubcoreMesh(core_axis_name="c", subcore_axis_name="s",
                                  num_cores=_NC, num_subcores=_NS)
    f = pl.kernel(
        _sc_flow_body,
        out_type=[jax.ShapeDtypeStruct((_NC, _B, _NPAD), jnp.float32),
                  jax.ShapeDtypeStruct((_NC, _B, _NPAD), jnp.float32)],
        mesh=mesh,
        scratch_types=[pltpu.VMEM_SHARED((_NPAD,), jnp.float32)] * 8
                      + [pltpu.VMEM((_K,), jnp.int32)] * 2
                      + [pltpu.VMEM((_K,), jnp.float32)] * 10
                      + [pltpu.VMEM((_SLICE,), jnp.float32)],
        interpret=_INTERPRET,
    )
    return f(a0, a1, b0, b1, srcx, dstx, g0, g1, s0, s1)


def _flow_partials(ab, edge_index, conductance, susceptance):
    a = ab[0, :, :_N]
    b = ab[1, :, :_N]
    src = edge_index[0]
    dst = edge_index[1]
    ai = jnp.take(a, src, axis=1)
    aj = jnp.take(a, dst, axis=1)
    bi = jnp.take(b, src, axis=1)
    bj = jnp.take(b, dst, axis=1)
    c = ai * aj + bi * bj
    s = bi * aj - ai * bj
    p_ij = conductance * c + susceptance * s
    q_ij = conductance * s - susceptance * c
    p_calc = jnp.zeros((_B, _N), jnp.float32)
    p_calc = p_calc.at[:, src].add(p_ij)
    p_calc = p_calc.at[:, dst].add(-p_ij)
    q_calc = jnp.zeros((_B, _N), jnp.float32)
    q_calc = q_calc.at[:, src].add(q_ij)
    q_calc = q_calc.at[:, dst].add(-q_ij)
    pad = ((0, 0), (0, _NPAD - _N))
    return (jnp.pad(p_calc, pad)[None], jnp.pad(q_calc, pad)[None])


def kernel(failure_probability, failure_label, voltages, angles, line_flows,
           frequency, target_voltages, conductance, susceptance,
           power_injection, thermal_limits, reactive_injection,
           power_imbalance, edge_index):
    v = voltages[..., 0]
    th = angles[..., 0]
    fp = failure_probability[..., 0]
    fl = failure_label[..., 0]
    tgt = target_voltages[..., 0]
    pinj = power_injection[..., 0]
    qinj = reactive_injection[..., 0]
    lf = line_flows[..., 0]

    pad = ((0, 0), (0, _NPAD - _N))
    vpad = jnp.pad(v, pad)
    apad = jnp.pad(th, pad)
    ab = _run_prep(vpad, apad)

    pp, qp = _run_sc_flow(ab[0, 0], ab[0, 1], ab[1, 0], ab[1, 1],
                          edge_index[0], edge_index[1],
                          conductance[0], conductance[1],
                          susceptance[0], susceptance[1])

    sums = _run_node_loss(fp, fl, v, tgt, pinj, qinj, pp, qp,
                          frequency, power_imbalance)
    cap = _run_edge_loss(lf, thermal_limits)

    bn = float(_B * _N)
    total = (sums[0] / bn
             + 0.1 * (sums[1] / bn)
             + 0.05 * (cap[0] / float(_B * _E))
             + 0.05 * (sums[3] / bn)
             + 0.08 * (sums[5] / float(_B))
             + 1.0 * (sums[4] / bn)
             + 0.1 * (sums[2] / bn))
    return total
